# Initial kernel scaffold; baseline (speedup 1.0000x reference)
import jax
import jax.numpy as jnp
from jax.experimental import pallas as pl


def kernel(verts_features, faces, pix_to_face, bary_coords):
    raise NotImplementedError("write your pallas kernel here")


# speedup vs baseline: 67.3664x; 67.3664x over previous
import functools

import jax
import jax.numpy as jnp
from jax import lax
from jax.experimental import pallas as pl
from jax.experimental.pallas import tpu as pltpu
from jax.experimental.pallas import tpu_sc as plsc

NW = 32
L = 16


def _wid():
    return lax.axis_index("s") * 2 + lax.axis_index("c")


def _face_table_body(RO, GS, verts_hbm, facesT_hbm, fcT_hbm, idx_v, vals_v, sem):
    wid = _wid()
    for k in range(3):
        pltpu.sync_copy(facesT_hbm.at[k, wid], idx_v)

        def group(g, carry):
            base = g * GS
            descs = [
                pltpu.make_async_copy(
                    verts_hbm.at[idx_v.at[base + t]], vals_v.at[base + t], sem)
                for t in range(GS)
            ]
            for d in descs:
                d.start()
            for d in descs:
                d.wait()
            return carry

        lax.fori_loop(0, RO // GS, group, 0)
        pltpu.sync_copy(vals_v, fcT_hbm.at[k, wid])


def _shade_body(PT, CH, fc0_hbm, fc1_hbm, fc2_hbm, p2f_hbm, b0_hbm, b1_hbm,
                b2_hbm, out_hbm, idx_v, g0_v, g1_v, g2_v, b0_v, b1_v, b2_v,
                out_v, sem):
    wid = _wid()
    NR = CH // 128
    fcs = (fc0_hbm, fc1_hbm, fc2_hbm)
    bhs = (b0_hbm, b1_hbm, b2_hbm)
    gs = (g0_v, g1_v, g2_v)
    bs = (b0_v, b1_v, b2_v)

    def chunk(c, carry):
        row0 = wid * (PT // 128) + c * NR
        base = wid * PT + c * CH
        pltpu.sync_copy(p2f_hbm.at[pl.ds(row0, NR)], idx_v)
        for k in range(3):
            pltpu.sync_copy(bhs[k].at[pl.ds(base, CH)], bs[k])
        descs = [
            pltpu.make_async_copy(
                fcs[k].at[idx_v.at[j]], gs[k].at[pl.ds(j * 128, 128)], sem)
            for k in range(3)
            for j in range(NR)
        ]
        for d in descs:
            d.start()
        for d in descs:
            d.wait()

        def pix(i, carry2):
            s = pl.ds(i * L, L)
            acc = gs[0][s] * bs[0][s]
            acc = acc + gs[1][s] * bs[1][s]
            acc = acc + gs[2][s] * bs[2][s]
            out_v[s] = acc.astype(jnp.int32)
            return carry2

        lax.fori_loop(0, CH // L, pix, 0)
        pltpu.sync_copy(out_v, out_hbm.at[pl.ds(base, CH)])
        return carry

    lax.fori_loop(0, PT // CH, chunk, 0)


def kernel(verts_features, faces, pix_to_face, bary_coords):
    N, V, C = verts_features.shape
    F = faces.shape[0]
    _, H, W, K = pix_to_face.shape
    P = N * H * W * K

    mesh = plsc.VectorSubcoreMesh(core_axis_name="c", subcore_axis_name="s")

    RO = -(-F // (NW * 128))
    GS = 7 if RO % 7 == 0 else 1
    Fp = NW * RO * 128
    verts_flat = verts_features.reshape(N * V)
    facesT = faces.T
    facesT = jnp.pad(facesT, ((0, 0), (0, Fp - F)))
    facesT = facesT.reshape(3, NW, RO, 128)

    face_table = pl.kernel(
        functools.partial(_face_table_body, RO, GS),
        mesh=mesh,
        out_type=jax.ShapeDtypeStruct((3, NW, RO, 128), jnp.float32),
        scratch_types=[
            pltpu.VMEM((RO, 128), jnp.int32),
            pltpu.VMEM((RO, 128), jnp.float32),
            pltpu.SemaphoreType.DMA,
        ],
    )
    fcT = face_table(verts_flat, facesT).reshape(3, Fp)

    PT = P // NW
    CH = 1024
    p2f = pix_to_face.reshape(P // 128, 128)
    baryT = bary_coords.reshape(P, 3).T

    shade = pl.kernel(
        functools.partial(_shade_body, PT, CH),
        mesh=mesh,
        out_type=jax.ShapeDtypeStruct((P,), jnp.int32),
        scratch_types=[
            pltpu.VMEM((CH // 128, 128), jnp.int32),
            pltpu.VMEM((CH,), jnp.float32),
            pltpu.VMEM((CH,), jnp.float32),
            pltpu.VMEM((CH,), jnp.float32),
            pltpu.VMEM((CH,), jnp.float32),
            pltpu.VMEM((CH,), jnp.float32),
            pltpu.VMEM((CH,), jnp.float32),
            pltpu.VMEM((CH,), jnp.int32),
            pltpu.SemaphoreType.DMA,
        ],
    )
    out = shade(fcT[0], fcT[1], fcT[2], p2f, baryT[0], baryT[1], baryT[2])
    return out.reshape(N, H, W, K)


# speedup vs baseline: 96.8697x; 1.4380x over previous
import functools

import jax
import jax.numpy as jnp
from jax import lax
from jax.experimental import pallas as pl
from jax.experimental.pallas import tpu as pltpu
from jax.experimental.pallas import tpu_sc as plsc

NW = 32
L = 16


def _wid():
    return lax.axis_index("s") * 2 + lax.axis_index("c")


def _face_table_body(RO, GS, verts_hbm, facesT_hbm, fcT_hbm, idx_v, vals_v, sem):
    wid = _wid()
    NG = RO // GS

    def fire(g):
        for t in range(GS):
            pltpu.make_async_copy(
                verts_hbm.at[idx_v.at[g * GS + t]],
                vals_v.at[g * GS + t], sem).start()

    def drain(g, k):
        pltpu.make_async_copy(
            fcT_hbm.at[k, wid, pl.ds(0, GS)],
            vals_v.at[pl.ds(g * GS, GS)], sem).wait()

    for k in range(3):
        pltpu.sync_copy(facesT_hbm.at[k, wid], idx_v)
        fire(0)

        def step(g, carry):
            fire(g + 1)
            drain(g, k)
            return carry

        lax.fori_loop(0, NG - 1, step, 0)
        drain(NG - 1, k)
        pltpu.sync_copy(vals_v, fcT_hbm.at[k, wid])


def _shade_body(PT, CH, fc0_hbm, fc1_hbm, fc2_hbm, p2f_hbm, b0_hbm, b1_hbm,
                b2_hbm, out_hbm,
                idx0_v, idx1_v,
                g00, g01, g02, g10, g11, g12,
                b00, b01, b02, b10, b11, b12,
                out0_v, out1_v,
                sin0, sin1, sg0, sg1, so0, so1):
    wid = _wid()
    NR = CH // 128
    NC = PT // CH
    fcs = (fc0_hbm, fc1_hbm, fc2_hbm)
    bhs = (b0_hbm, b1_hbm, b2_hbm)
    idxs = (idx0_v, idx1_v)
    gss = ((g00, g01, g02), (g10, g11, g12))
    bss = ((b00, b01, b02), (b10, b11, b12))
    outs = (out0_v, out1_v)
    sins = (sin0, sin1)
    sgs = (sg0, sg1)
    sos = (so0, so1)

    def in_descs(c, p):
        row0 = wid * (PT // 128) + c * NR
        base = wid * PT + c * CH
        ds = [pltpu.make_async_copy(
            p2f_hbm.at[pl.ds(row0, NR)], idxs[p], sins[p])]
        for k in range(3):
            ds.append(pltpu.make_async_copy(
                bhs[k].at[pl.ds(base, CH)], bss[p][k], sins[p]))
        return ds

    def start_in(c, p):
        for d in in_descs(c, p):
            d.start()

    def wait_in(c, p):
        for d in in_descs(c, p):
            d.wait()

    def fire_g(p):
        for k in range(3):
            for j in range(NR):
                pltpu.make_async_copy(
                    fcs[k].at[idxs[p].at[j]],
                    gss[p][k].at[pl.ds(j * 128, 128)], sgs[p]).start()

    def drain_g(p):
        for k in range(3):
            pltpu.make_async_copy(
                fcs[k].at[pl.ds(0, CH)], gss[p][k], sgs[p]).wait()

    def out_desc(c, p):
        base = wid * PT + c * CH
        return pltpu.make_async_copy(
            outs[p], out_hbm.at[pl.ds(base, CH)], sos[p])

    def compute(p):
        gs, bs, out_v = gss[p], bss[p], outs[p]

        def pix(i, carry2):
            s = pl.ds(i * L, L)
            acc = gs[0][s] * bs[0][s]
            acc = acc + gs[1][s] * bs[1][s]
            acc = acc + gs[2][s] * bs[2][s]
            out_v[s] = acc.astype(jnp.int32)
            return carry2

        lax.fori_loop(0, CH // L, pix, 0)

    def step(c, p, first, last2):
        if not last2:
            wait_in(c + 1, 1 - p)
            fire_g(1 - p)
        elif c + 1 < NC:
            pass
        drain_g(p)
        if not first:
            out_desc(c - 2, p).wait()
        compute(p)
        out_desc(c, p).start()
        if not last2:
            start_in(c + 2, p)

    start_in(0, 0)
    wait_in(0, 0)
    fire_g(0)
    start_in(1, 1)

    def loop_body(t, carry):
        c = 2 * t
        step(c, 0, False, False)
        step(c + 1, 1, False, False)
        return carry

    step(0, 0, True, False)
    step(1, 1, True, False)
    lax.fori_loop(1, NC // 2 - 1, loop_body, 0)

    c = NC - 2
    wait_in(c + 1, 1)
    fire_g(1)
    drain_g(0)
    out_desc(c - 2, 0).wait()
    compute(0)
    out_desc(c, 0).start()

    c = NC - 1
    drain_g(1)
    out_desc(c - 2, 1).wait()
    compute(1)
    out_desc(c, 1).start()

    out_desc(NC - 2, 0).wait()
    out_desc(NC - 1, 1).wait()


def kernel(verts_features, faces, pix_to_face, bary_coords):
    N, V, C = verts_features.shape
    F = faces.shape[0]
    _, H, W, K = pix_to_face.shape
    P = N * H * W * K

    mesh = plsc.VectorSubcoreMesh(core_axis_name="c", subcore_axis_name="s")

    RO = -(-F // (NW * 128))
    GS = 7 if RO % 7 == 0 else 1
    Fp = NW * RO * 128
    verts_flat = verts_features.reshape(N * V)
    facesT = faces.T
    facesT = jnp.pad(facesT, ((0, 0), (0, Fp - F)))
    facesT = facesT.reshape(3, NW, RO, 128)

    face_table = pl.kernel(
        functools.partial(_face_table_body, RO, GS),
        mesh=mesh,
        out_type=jax.ShapeDtypeStruct((3, NW, RO, 128), jnp.float32),
        scratch_types=[
            pltpu.VMEM((RO, 128), jnp.int32),
            pltpu.VMEM((RO, 128), jnp.float32),
            pltpu.SemaphoreType.DMA,
        ],
    )
    fcT = face_table(verts_flat, facesT).reshape(3, Fp)

    PT = P // NW
    CH = 1024
    p2f = pix_to_face.reshape(P // 128, 128)
    baryT = bary_coords.reshape(P, 3).T

    shade = pl.kernel(
        functools.partial(_shade_body, PT, CH),
        mesh=mesh,
        out_type=jax.ShapeDtypeStruct((P,), jnp.int32),
        scratch_types=(
            [pltpu.VMEM((CH // 128, 128), jnp.int32)] * 2
            + [pltpu.VMEM((CH,), jnp.float32)] * 6
            + [pltpu.VMEM((CH,), jnp.float32)] * 6
            + [pltpu.VMEM((CH,), jnp.int32)] * 2
            + [pltpu.SemaphoreType.DMA] * 6
        ),
    )
    out = shade(fcT[0], fcT[1], fcT[2], p2f, baryT[0], baryT[1], baryT[2])
    return out.reshape(N, H, W, K)
